# baseline (speedup 1.0000x reference)
import jax
import jax.numpy as jnp
from jax.experimental import pallas as pl
from jax.experimental.pallas import tpu as pltpu

N_NODES = 10000
D_IN = 128
D_OUT = 128
CM = 400
NCHUNK = N_NODES // CM
NBUF = 3


def _gcn_pipelined_kernel(a_hbm, x_ref, wt_ref, b_ref, o_ref, abuf, sem):
    def start_copy(slot, c):
        pltpu.make_async_copy(
            a_hbm.at[pl.ds(c * CM, CM), :], abuf.at[slot], sem.at[slot]
        ).start()

    for s in range(NBUF):
        start_copy(s, s)

    def step(c, carry):
        slot = jax.lax.rem(c, NBUF)
        pltpu.make_async_copy(
            a_hbm.at[pl.ds(c * CM, CM), :], abuf.at[slot], sem.at[slot]
        ).wait()
        a_bf = abuf[slot].astype(jnp.bfloat16)
        support = jnp.dot(a_bf, x_ref[...], preferred_element_type=jnp.float32)
        o_ref[pl.ds(c * CM, CM), :] = (
            jnp.dot(support, wt_ref[...], preferred_element_type=jnp.float32)
            + b_ref[...]
        )
        nxt = c + NBUF

        @pl.when(nxt < NCHUNK)
        def _():
            start_copy(slot, nxt)

        return carry

    jax.lax.fori_loop(0, NCHUNK, step, 0)


def kernel(x, adj_normalized, W, b):
    x = x.astype(jnp.bfloat16)
    wt = W.T
    b2 = b.reshape(1, D_OUT)
    out = pl.pallas_call(
        _gcn_pipelined_kernel,
        in_specs=[
            pl.BlockSpec(memory_space=pl.ANY),
            pl.BlockSpec((N_NODES, D_IN), lambda: (0, 0)),
            pl.BlockSpec((D_IN, D_OUT), lambda: (0, 0)),
            pl.BlockSpec((1, D_OUT), lambda: (0, 0)),
        ],
        out_specs=pl.BlockSpec((N_NODES, D_OUT), lambda: (0, 0)),
        out_shape=jax.ShapeDtypeStruct((N_NODES, D_OUT), jnp.float32),
        scratch_shapes=[
            pltpu.VMEM((NBUF, CM, N_NODES), jnp.float32),
            pltpu.SemaphoreType.DMA((NBUF,)),
        ],
        compiler_params=pltpu.CompilerParams(vmem_limit_bytes=64 * 1024 * 1024),
    )(adj_normalized, x, wt, b2)
    return out


# speedup vs baseline: 1.0737x; 1.0737x over previous
import jax
import jax.numpy as jnp
from jax import lax
from jax.experimental import pallas as pl
from jax.experimental.pallas import tpu as pltpu

N_NODES = 10000
D_IN = 128
D_OUT = 128
BM = 400


def _gcn_block_kernel(a_ref, x_ref, w_ref, b_ref, o_ref, xbf_ref):
    @pl.when(pl.program_id(0) == 0)
    def _():
        xbf_ref[...] = x_ref[...].astype(jnp.bfloat16)

    a_bf = a_ref[...].astype(jnp.bfloat16)
    support = jnp.dot(a_bf, xbf_ref[...], preferred_element_type=jnp.float32)
    out = lax.dot_general(
        support, w_ref[...], (((1,), (1,)), ((), ())),
        preferred_element_type=jnp.float32,
    )
    o_ref[...] = out + b_ref[...]


def kernel(x, adj_normalized, W, b):
    b2 = b.reshape(1, D_OUT)
    grid = (N_NODES // BM,)
    out = pl.pallas_call(
        _gcn_block_kernel,
        grid=grid,
        in_specs=[
            pl.BlockSpec((BM, N_NODES), lambda i: (i, 0)),
            pl.BlockSpec((N_NODES, D_IN), lambda i: (0, 0)),
            pl.BlockSpec((D_OUT, D_IN), lambda i: (0, 0)),
            pl.BlockSpec((1, D_OUT), lambda i: (0, 0)),
        ],
        out_specs=pl.BlockSpec((BM, D_OUT), lambda i: (i, 0)),
        out_shape=jax.ShapeDtypeStruct((N_NODES, D_OUT), jnp.float32),
        scratch_shapes=[pltpu.VMEM((N_NODES, D_IN), jnp.bfloat16)],
        compiler_params=pltpu.CompilerParams(vmem_limit_bytes=60 * 1024 * 1024),
    )(adj_normalized, x, W, b2)
    return out
